# baseline (speedup 1.0000x reference)
import functools

import jax
import jax.numpy as jnp
from jax import lax
from jax.experimental import pallas as pl
from jax.experimental.pallas import tpu as pltpu
from jax.experimental.pallas import tpu_sc as plsc

N = 10000
E = 320000
D = 128

NC = 2
NS = 16
NW = NC * NS

BATCH = 40
NBUF = 8
NB = E // (NW * BATCH)
N_PAD = 10112
ZROWS = N_PAD // NS

MM_BLK = 1000


def _i32(v):
    return jnp.asarray(v, jnp.int32)


def _mmc_body(p_ref, w_ref, b_ref, o_ref):
    agg = p_ref[0] + p_ref[1]
    o_ref[...] = jnp.dot(agg, w_ref[...],
                         preferred_element_type=jnp.float32) + b_ref[...]


def _matmul_combine(parts, w, b2d):
    return pl.pallas_call(
        _mmc_body,
        grid=(N // MM_BLK,),
        in_specs=[
            pl.BlockSpec((2, MM_BLK, D), lambda i: (_i32(0), i, _i32(0))),
            pl.BlockSpec((D, D), lambda i: (_i32(0), _i32(0))),
            pl.BlockSpec((1, D), lambda i: (_i32(0), _i32(0))),
        ],
        out_specs=pl.BlockSpec((MM_BLK, D), lambda i: (i, _i32(0))),
        out_shape=jax.ShapeDtypeStruct((N, D), jnp.float32),
    )(parts, w, b2d)


@functools.partial(
    pl.kernel,
    mesh=plsc.VectorSubcoreMesh(core_axis_name="c", subcore_axis_name="s"),
    out_type=jax.ShapeDtypeStruct((NC, N_PAD, D), jnp.float32),
    scratch_types=[
        pltpu.VMEM((NBUF, BATCH), jnp.int32),
        pltpu.VMEM((NBUF, BATCH), jnp.int32),
        pltpu.VMEM((NBUF, BATCH, D), jnp.float32),
        pltpu.VMEM_SHARED((N_PAD, D), jnp.float32),
        pltpu.SemaphoreType.DMA,
        pltpu.SemaphoreType.DMA,
    ],
)
def _sc_scatter(support_hbm, ei_hbm, zeros_hbm, out_hbm,
                src_v, dst_v, rows_v, acc_sh, sem_g, sem_i):
    c = lax.axis_index("c")
    s = lax.axis_index("s")
    wid = s * NC + c
    base = wid * (NB * BATCH)
    i0 = jnp.int32(0)
    i1 = jnp.int32(1)
    i2 = jnp.int32(2)

    def src_at(j):
        return ei_hbm.at[pl.ds(base + j * BATCH, BATCH)]

    def dst_at(j):
        return ei_hbm.at[pl.ds(E + base + j * BATCH, BATCH)]

    for k in range(NBUF - 1):
        ik = jnp.int32(k)
        pltpu.sync_copy(src_at(ik), src_v.at[ik])
        pltpu.async_copy(support_hbm.at[src_v.at[ik]], rows_v.at[ik], sem_g)
    for k in range(NBUF - 1):
        ik = jnp.int32(k)
        pltpu.sync_copy(dst_at(ik), dst_v.at[ik])
    ilast = jnp.int32(NBUF - 1)
    pltpu.async_copy(src_at(ilast), src_v.at[ilast], sem_i)
    pltpu.async_copy(dst_at(ilast), dst_v.at[ilast], sem_i)

    pltpu.sync_copy(zeros_hbm, acc_sh.at[pl.ds(s * ZROWS, ZROWS)])
    plsc.subcore_barrier()

    def step(j, carry):
        p = j % NBUF
        pltpu.make_async_copy(support_hbm.at[src_v.at[p]],
                              rows_v.at[p], sem_g).wait()
        pltpu.sync_copy(rows_v.at[p], acc_sh.at[dst_v.at[p]], add=True)

        @pl.when(j + NBUF - 1 < NB)
        def _():
            q = (j + NBUF - 1) % NBUF
            pltpu.make_async_copy(src_at(j + NBUF - 1), src_v.at[q],
                                  sem_i).wait()
            pltpu.make_async_copy(dst_at(j + NBUF - 1), dst_v.at[q],
                                  sem_i).wait()
            pltpu.async_copy(support_hbm.at[src_v.at[q]], rows_v.at[q],
                             sem_g)

        @pl.when(j + NBUF < NB)
        def _():
            pltpu.async_copy(src_at(j + NBUF), src_v.at[p], sem_i)
            pltpu.async_copy(dst_at(j + NBUF), dst_v.at[p], sem_i)

        return carry

    lax.fori_loop(jnp.int32(0), jnp.int32(NB), step, jnp.int32(0))

    plsc.subcore_barrier()
    pltpu.sync_copy(acc_sh.at[pl.ds(s * ZROWS, ZROWS)],
                    out_hbm.at[c, pl.ds(s * ZROWS, ZROWS)])


def kernel(x, edge_index, W, b):
    ei32 = edge_index.astype(jnp.int32).reshape(-1)
    zeros = jnp.zeros((ZROWS, D), jnp.float32)

    parts = _sc_scatter(x, ei32, zeros)
    return _matmul_combine(parts, W, b.reshape(1, D))


# speedup vs baseline: 1.3000x; 1.3000x over previous
import functools

import jax
import jax.numpy as jnp
from jax import lax
from jax.experimental import pallas as pl
from jax.experimental.pallas import tpu as pltpu
from jax.experimental.pallas import tpu_sc as plsc

N = 10000
E = 320000
D = 128

NC = 2
NS = 16
NW = NC * NS

BATCH = 80
NBUF = 4
IB = NBUF + 1
NB = E // (NW * BATCH)
N_PAD = 10112
ZROWS = N_PAD // NS

MM_BLK = 1000


def _i32(v):
    return jnp.asarray(v, jnp.int32)


def _mmc_body(p_ref, w_ref, b_ref, o_ref):
    agg = p_ref[0] + p_ref[1]
    o_ref[...] = jnp.dot(agg, w_ref[...],
                         preferred_element_type=jnp.float32) + b_ref[...]


def _matmul_combine(parts, w, b2d):
    return pl.pallas_call(
        _mmc_body,
        grid=(N // MM_BLK,),
        in_specs=[
            pl.BlockSpec((2, MM_BLK, D), lambda i: (_i32(0), i, _i32(0))),
            pl.BlockSpec((D, D), lambda i: (_i32(0), _i32(0))),
            pl.BlockSpec((1, D), lambda i: (_i32(0), _i32(0))),
        ],
        out_specs=pl.BlockSpec((MM_BLK, D), lambda i: (i, _i32(0))),
        out_shape=jax.ShapeDtypeStruct((N, D), jnp.float32),
    )(parts, w, b2d)


@functools.partial(
    pl.kernel,
    mesh=plsc.VectorSubcoreMesh(core_axis_name="c", subcore_axis_name="s"),
    out_type=jax.ShapeDtypeStruct((NC, N_PAD, D), jnp.float32),
    scratch_types=[
        pltpu.VMEM((IB, BATCH), jnp.int32),
        pltpu.VMEM((IB, BATCH), jnp.int32),
        pltpu.VMEM((NBUF, BATCH, D), jnp.float32),
        pltpu.VMEM_SHARED((N_PAD, D), jnp.float32),
        pltpu.SemaphoreType.DMA,
        pltpu.SemaphoreType.DMA,
        pltpu.SemaphoreType.DMA,
    ],
)
def _sc_scatter(support_hbm, ei_hbm, zeros_hbm, out_hbm,
                src_v, dst_v, rows_v, acc_sh, sem_g, sem_i, sem_s):
    c = lax.axis_index("c")
    s = lax.axis_index("s")
    wid = s * NC + c
    base = wid * (NB * BATCH)
    i0 = jnp.int32(0)
    i1 = jnp.int32(1)
    i2 = jnp.int32(2)

    def src_at(j):
        return ei_hbm.at[pl.ds(base + j * BATCH, BATCH)]

    def dst_at(j):
        return ei_hbm.at[pl.ds(E + base + j * BATCH, BATCH)]

    for k in range(NBUF - 1):
        ik = jnp.int32(k)
        pltpu.sync_copy(src_at(ik), src_v.at[ik])
        pltpu.async_copy(support_hbm.at[src_v.at[ik]], rows_v.at[ik], sem_g)
    for k in range(NBUF - 1):
        ik = jnp.int32(k)
        pltpu.sync_copy(dst_at(ik), dst_v.at[ik])
    ilast = jnp.int32(NBUF - 1)
    pltpu.async_copy(src_at(ilast), src_v.at[ilast], sem_i)
    pltpu.async_copy(dst_at(ilast), dst_v.at[ilast], sem_i)

    pltpu.sync_copy(zeros_hbm, acc_sh.at[pl.ds(s * ZROWS, ZROWS)])
    plsc.subcore_barrier()

    def step(j, carry):
        p = _i32(j % NBUF)
        pi = _i32(j % IB)

        @pl.when(j >= 1)
        def _():
            pltpu.make_async_copy(rows_v.at[_i32((j + NBUF - 1) % NBUF)],
                                  acc_sh.at[dst_v.at[_i32((j + IB - 1) % IB)]],
                                  sem_s).wait()

        @pl.when(j + NBUF < NB)
        def _():
            qi = _i32((j + NBUF) % IB)
            pltpu.async_copy(src_at(j + NBUF), src_v.at[qi], sem_i)
            pltpu.async_copy(dst_at(j + NBUF), dst_v.at[qi], sem_i)

        pltpu.make_async_copy(support_hbm.at[src_v.at[pi]],
                              rows_v.at[p], sem_g).wait()
        pltpu.async_copy(rows_v.at[p], acc_sh.at[dst_v.at[pi]], sem_s,
                         add=True)

        @pl.when(j + NBUF - 1 < NB)
        def _():
            qi = _i32((j + NBUF - 1) % IB)
            pltpu.make_async_copy(src_at(j + NBUF - 1), src_v.at[qi],
                                  sem_i).wait()
            pltpu.make_async_copy(dst_at(j + NBUF - 1), dst_v.at[qi],
                                  sem_i).wait()
            pltpu.async_copy(support_hbm.at[src_v.at[qi]],
                             rows_v.at[_i32((j + NBUF - 1) % NBUF)], sem_g)

        return carry

    lax.fori_loop(jnp.int32(0), jnp.int32(NB), step, jnp.int32(0))

    pltpu.make_async_copy(rows_v.at[_i32((NB - 1) % NBUF)],
                          acc_sh.at[dst_v.at[_i32((NB - 1) % IB)]],
                          sem_s).wait()
    plsc.subcore_barrier()
    pltpu.sync_copy(acc_sh.at[pl.ds(s * ZROWS, ZROWS)],
                    out_hbm.at[c, pl.ds(s * ZROWS, ZROWS)])


def kernel(x, edge_index, W, b):
    ei32 = edge_index.astype(jnp.int32).reshape(-1)
    zeros = jnp.zeros((ZROWS, D), jnp.float32)

    parts = _sc_scatter(x, ei32, zeros)
    return _matmul_combine(parts, W, b.reshape(1, D))


# speedup vs baseline: 1.3229x; 1.0176x over previous
import functools

import jax
import jax.numpy as jnp
from jax import lax
from jax.experimental import pallas as pl
from jax.experimental.pallas import tpu as pltpu
from jax.experimental.pallas import tpu_sc as plsc

N = 10000
E = 320000
D = 128

NC = 2
NS = 16
NW = NC * NS

BATCH = 80
NBUF = 4
NB = E // (NW * BATCH)
N_PAD = 10112
ZROWS = N_PAD // NS

MM_BLK = 1000


def _i32(v):
    return jnp.asarray(v, jnp.int32)


def _mmc_body(p_ref, w_ref, b_ref, o_ref):
    agg = p_ref[0] + p_ref[1]
    o_ref[...] = jnp.dot(agg, w_ref[...],
                         preferred_element_type=jnp.float32) + b_ref[...]


def _matmul_combine(parts, w, b2d):
    return pl.pallas_call(
        _mmc_body,
        grid=(N // MM_BLK,),
        in_specs=[
            pl.BlockSpec((2, MM_BLK, D), lambda i: (_i32(0), i, _i32(0))),
            pl.BlockSpec((D, D), lambda i: (_i32(0), _i32(0))),
            pl.BlockSpec((1, D), lambda i: (_i32(0), _i32(0))),
        ],
        out_specs=pl.BlockSpec((MM_BLK, D), lambda i: (i, _i32(0))),
        out_shape=jax.ShapeDtypeStruct((N, D), jnp.float32),
    )(parts, w, b2d)


@functools.partial(
    pl.kernel,
    mesh=plsc.VectorSubcoreMesh(core_axis_name="c", subcore_axis_name="s"),
    out_type=jax.ShapeDtypeStruct((NC, N_PAD, D), jnp.float32),
    scratch_types=[
        pltpu.VMEM((NBUF, BATCH), jnp.int32),
        pltpu.VMEM((NBUF, BATCH), jnp.int32),
        pltpu.VMEM((NBUF, BATCH, D), jnp.float32),
        pltpu.VMEM_SHARED((N_PAD, D), jnp.float32),
        pltpu.SemaphoreType.DMA,
        pltpu.SemaphoreType.DMA,
    ],
)
def _sc_scatter(support_hbm, ei_hbm, zeros_hbm, out_hbm,
                src_v, dst_v, rows_v, acc_sh, sem_g, sem_i):
    c = lax.axis_index("c")
    s = lax.axis_index("s")
    wid = s * NC + c
    base = wid * (NB * BATCH)
    i0 = jnp.int32(0)
    i1 = jnp.int32(1)
    i2 = jnp.int32(2)

    def src_at(j):
        return ei_hbm.at[pl.ds(base + j * BATCH, BATCH)]

    def dst_at(j):
        return ei_hbm.at[pl.ds(E + base + j * BATCH, BATCH)]

    for k in range(NBUF - 1):
        ik = jnp.int32(k)
        pltpu.sync_copy(src_at(ik), src_v.at[ik])
        pltpu.async_copy(support_hbm.at[src_v.at[ik]], rows_v.at[ik], sem_g)
    for k in range(NBUF - 1):
        ik = jnp.int32(k)
        pltpu.sync_copy(dst_at(ik), dst_v.at[ik])
    ilast = jnp.int32(NBUF - 1)
    pltpu.async_copy(src_at(ilast), src_v.at[ilast], sem_i)
    pltpu.async_copy(dst_at(ilast), dst_v.at[ilast], sem_i)

    pltpu.sync_copy(zeros_hbm, acc_sh.at[pl.ds(s * ZROWS, ZROWS)])
    plsc.subcore_barrier()

    def step(j, carry):
        p = _i32(j % NBUF)
        pltpu.make_async_copy(support_hbm.at[src_v.at[p]],
                              rows_v.at[p], sem_g).wait()
        pltpu.sync_copy(rows_v.at[p], acc_sh.at[dst_v.at[p]], add=True)

        @pl.when(j + NBUF - 1 < NB)
        def _():
            q = _i32((j + NBUF - 1) % NBUF)
            pltpu.make_async_copy(src_at(j + NBUF - 1), src_v.at[q],
                                  sem_i).wait()
            pltpu.make_async_copy(dst_at(j + NBUF - 1), dst_v.at[q],
                                  sem_i).wait()
            pltpu.async_copy(support_hbm.at[src_v.at[q]], rows_v.at[q],
                             sem_g)

        @pl.when(j + NBUF < NB)
        def _():
            pltpu.async_copy(src_at(j + NBUF), src_v.at[p], sem_i)
            pltpu.async_copy(dst_at(j + NBUF), dst_v.at[p], sem_i)

        return carry

    lax.fori_loop(jnp.int32(0), jnp.int32(NB), step, jnp.int32(0))

    plsc.subcore_barrier()
    pltpu.sync_copy(acc_sh.at[pl.ds(s * ZROWS, ZROWS)],
                    out_hbm.at[c, pl.ds(s * ZROWS, ZROWS)])


def kernel(x, edge_index, W, b):
    ei32 = edge_index.astype(jnp.int32).reshape(-1)
    zeros = jnp.zeros((ZROWS, D), jnp.float32)

    parts = _sc_scatter(x, ei32, zeros)
    return _matmul_combine(parts, W, b.reshape(1, D))


# speedup vs baseline: 1.3232x; 1.0002x over previous
import functools

import jax
import jax.numpy as jnp
from jax import lax
from jax.experimental import pallas as pl
from jax.experimental.pallas import tpu as pltpu
from jax.experimental.pallas import tpu_sc as plsc

N = 10000
E = 320000
D = 128

NC = 2
NS = 16
NW = NC * NS

BATCH = 80
NBUF = 4
NB = E // (NW * BATCH)
N_PAD = 10112
ZROWS = N_PAD // NS

MM_BLK = 1000


def _i32(v):
    return jnp.asarray(v, jnp.int32)


def _mmc_body(p_ref, w_ref, b_ref, o_ref):
    agg = p_ref[0] + p_ref[1]
    o_ref[...] = jnp.dot(agg, w_ref[...],
                         preferred_element_type=jnp.float32) + b_ref[...]


def _matmul_combine(parts, w, b2d):
    return pl.pallas_call(
        _mmc_body,
        grid=(N // MM_BLK,),
        in_specs=[
            pl.BlockSpec((2, MM_BLK, D), lambda i: (_i32(0), i, _i32(0))),
            pl.BlockSpec((D, D), lambda i: (_i32(0), _i32(0))),
            pl.BlockSpec((1, D), lambda i: (_i32(0), _i32(0))),
        ],
        out_specs=pl.BlockSpec((MM_BLK, D), lambda i: (i, _i32(0))),
        out_shape=jax.ShapeDtypeStruct((N, D), jnp.float32),
    )(parts, w, b2d)


@functools.partial(
    pl.kernel,
    mesh=plsc.VectorSubcoreMesh(core_axis_name="c", subcore_axis_name="s"),
    out_type=jax.ShapeDtypeStruct((NC, N_PAD, D), jnp.float32),
    scratch_types=[
        pltpu.VMEM((NBUF, BATCH), jnp.int32),
        pltpu.VMEM((NBUF, BATCH), jnp.int32),
        pltpu.VMEM((NBUF, BATCH, D), jnp.float32),
        pltpu.VMEM_SHARED((N_PAD, D), jnp.float32),
        pltpu.SemaphoreType.DMA,
        pltpu.SemaphoreType.DMA,
    ],
)
def _sc_scatter(support_hbm, ei_hbm, zeros_hbm, out_hbm,
                src_v, dst_v, rows_v, acc_sh, sem_g, sem_i):
    c = lax.axis_index("c")
    s = lax.axis_index("s")
    wid = s * NC + c
    base = wid * (NB * BATCH)

    def src_at(j):
        return ei_hbm.at[pl.ds(base + j * BATCH, BATCH)]

    def dst_at(j):
        return ei_hbm.at[pl.ds(E + base + j * BATCH, BATCH)]

    for k in range(NBUF - 1):
        ik = jnp.int32(k)
        pltpu.sync_copy(src_at(ik), src_v.at[ik])
        pltpu.async_copy(support_hbm.at[src_v.at[ik]], rows_v.at[ik], sem_g)
    for k in range(NBUF - 1):
        ik = jnp.int32(k)
        pltpu.sync_copy(dst_at(ik), dst_v.at[ik])
    ilast = jnp.int32(NBUF - 1)
    pltpu.async_copy(src_at(ilast), src_v.at[ilast], sem_i)
    pltpu.async_copy(dst_at(ilast), dst_v.at[ilast], sem_i)

    pltpu.sync_copy(zeros_hbm, acc_sh.at[pl.ds(s * ZROWS, ZROWS)])
    plsc.subcore_barrier()

    def step(j, carry):
        p = _i32(j % NBUF)
        pltpu.make_async_copy(support_hbm.at[src_v.at[p]],
                              rows_v.at[p], sem_g).wait()
        pltpu.sync_copy(rows_v.at[p], acc_sh.at[dst_v.at[p]], add=True)

        @pl.when(j + NBUF - 1 < NB)
        def _():
            q = _i32((j + NBUF - 1) % NBUF)
            pltpu.make_async_copy(src_at(j + NBUF - 1), src_v.at[q],
                                  sem_i).wait()
            pltpu.make_async_copy(dst_at(j + NBUF - 1), dst_v.at[q],
                                  sem_i).wait()
            pltpu.async_copy(support_hbm.at[src_v.at[q]], rows_v.at[q],
                             sem_g)

        @pl.when(j + NBUF < NB)
        def _():
            pltpu.async_copy(src_at(j + NBUF), src_v.at[p], sem_i)
            pltpu.async_copy(dst_at(j + NBUF), dst_v.at[p], sem_i)

        return carry

    lax.fori_loop(jnp.int32(0), jnp.int32(NB), step, jnp.int32(0))

    plsc.subcore_barrier()
    pltpu.sync_copy(acc_sh.at[pl.ds(s * ZROWS, ZROWS)],
                    out_hbm.at[c, pl.ds(s * ZROWS, ZROWS)])


def kernel(x, edge_index, W, b):
    ei32 = edge_index.astype(jnp.int32).reshape(-1)
    zeros = jnp.zeros((ZROWS, D), jnp.float32)

    parts = _sc_scatter(x, ei32, zeros)
    return _matmul_combine(parts, W, b.reshape(1, D))
